# baseline (speedup 1.0000x reference)
import functools

import jax
import jax.numpy as jnp
from jax import lax
from jax.experimental import pallas as pl
from jax.experimental.pallas import tpu as pltpu
from jax.experimental.pallas import tpu_sc as plsc

VOCAB = 100000
EMBED = 32
B = 1024
L = 200

NC = 2
NS = 16
NW = NC * NS
BPW = B // NW
CHUNK = 100
CPR = L // CHUNK
NCHUNK = BPW * CPR


LANE = 128


def _make_pool():
    mesh = plsc.VectorSubcoreMesh(core_axis_name="c", subcore_axis_name="s")

    @functools.partial(
        pl.kernel,
        mesh=mesh,
        compiler_params=pltpu.CompilerParams(use_tc_tiling_on_sc=True),
        out_type=jax.ShapeDtypeStruct((B, EMBED), jnp.float32),
        scratch_types=[
            pltpu.VMEM((NCHUNK, CHUNK), jnp.int32),
            pltpu.VMEM((CHUNK, LANE), jnp.float32),
            pltpu.VMEM((BPW, EMBED), jnp.float32),
            pltpu.SemaphoreType.DMA,
        ],
    )
    def pool(ids_hbm, table_hbm, out_hbm, idx_v, rows_v, out_v, sem):
        wid = lax.axis_index("s") * NC + lax.axis_index("c")
        pltpu.sync_copy(ids_hbm.at[wid], idx_v)

        def row_body(i, carry):
            def chunk_body(k, accs):
                a0, a1 = accs
                pltpu.async_copy(
                    table_hbm.at[idx_v.at[i * CPR + k]], rows_v, sem
                ).wait()

                def tok_body(t, accs2):
                    b0, b1 = accs2
                    return (b0 + rows_v[t, pl.ds(0, 16)],
                            b1 + rows_v[t, pl.ds(16, 16)])

                return lax.fori_loop(0, CHUNK, tok_body, (a0, a1), unroll=10)

            z = jnp.zeros((16,), jnp.float32)
            a0, a1 = lax.fori_loop(0, CPR, chunk_body, (z, z))
            out_v[i, pl.ds(0, 16)] = a0
            out_v[i, pl.ds(16, 16)] = a1
            return carry

        lax.fori_loop(0, BPW, row_body, 0)
        pltpu.sync_copy(out_v, out_hbm.at[pl.ds(wid * BPW, BPW)])

    return pool


_pool = _make_pool()

PAD_RB = 10000


def _pad_body(x_ref, o_ref):
    o_ref[...] = jnp.pad(x_ref[...], ((0, 0), (0, LANE - EMBED)))


def _pad_table(t):
    return pl.pallas_call(
        _pad_body,
        grid=(VOCAB // PAD_RB,),
        in_specs=[pl.BlockSpec((PAD_RB, EMBED), lambda i: (i, 0))],
        out_specs=pl.BlockSpec((PAD_RB, LANE), lambda i: (i, 0)),
        out_shape=jax.ShapeDtypeStruct((VOCAB, LANE), jnp.float32),
    )(t)


BN = 2048
GRID_N = (VOCAB + BN - 1) // BN


def _matmul_body(x_ref, w_ref, b_ref, o_ref):
    x = x_ref[...] * (1.0 / L)
    o_ref[...] = (
        jnp.dot(x, w_ref[...], preferred_element_type=jnp.float32) + b_ref[...]
    )


def _matmul(pooled, w, b2):
    return pl.pallas_call(
        _matmul_body,
        grid=(GRID_N,),
        in_specs=[
            pl.BlockSpec((B, EMBED), lambda n: (0, 0)),
            pl.BlockSpec((EMBED, BN), lambda n: (0, n)),
            pl.BlockSpec((1, BN), lambda n: (0, n)),
        ],
        out_specs=pl.BlockSpec((B, BN), lambda n: (0, n)),
        out_shape=jax.ShapeDtypeStruct((B, VOCAB), jnp.float32),
    )(pooled, w, b2)


def kernel(input_ids, embed_table, W, b):
    ids3 = input_ids.reshape(NW, NCHUNK, CHUNK)
    table128 = _pad_table(embed_table)
    pooled = _pool(ids3, table128)
    logits = _matmul(pooled, W, b.reshape(1, VOCAB))
    return logits[:, None, :]


# speedup vs baseline: 2.2887x; 2.2887x over previous
import functools

import jax
import jax.numpy as jnp
from jax import lax
from jax.experimental import pallas as pl
from jax.experimental.pallas import tpu as pltpu
from jax.experimental.pallas import tpu_sc as plsc

VOCAB = 100000
EMBED = 32
B = 1024
L = 200

NC = 2
NS = 16
NW = NC * NS
BPW = B // NW
CHUNK = 100
CPR = L // CHUNK
NCHUNK = BPW * CPR


def _make_pool():
    mesh = plsc.VectorSubcoreMesh(core_axis_name="c", subcore_axis_name="s")

    @functools.partial(
        pl.kernel,
        mesh=mesh,
        compiler_params=pltpu.CompilerParams(use_tc_tiling_on_sc=False),
        out_type=jax.ShapeDtypeStruct((B, EMBED), jnp.float32),
        scratch_types=[
            pltpu.VMEM((NCHUNK, CHUNK), jnp.int32),
            pltpu.VMEM((CHUNK, EMBED), jnp.float32),
            pltpu.VMEM((BPW, EMBED), jnp.float32),
            pltpu.SemaphoreType.DMA,
        ],
    )
    def pool(ids_hbm, table_hbm, out_hbm, idx_v, rows_v, out_v, sem):
        wid = lax.axis_index("s") * NC + lax.axis_index("c")
        pltpu.sync_copy(ids_hbm.at[wid], idx_v)
        inv_l = jnp.float32(1.0 / L)

        def row_body(i, carry):
            def chunk_body(k, accs):
                a0, a1 = accs
                pltpu.async_copy(
                    table_hbm.at[idx_v.at[i * CPR + k]], rows_v, sem
                ).wait()

                def tok_body(t, accs2):
                    b0, b1 = accs2
                    return (b0 + rows_v[t, pl.ds(0, 16)],
                            b1 + rows_v[t, pl.ds(16, 16)])

                return lax.fori_loop(0, CHUNK, tok_body, (a0, a1), unroll=10)

            z = jnp.zeros((16,), jnp.float32)
            a0, a1 = lax.fori_loop(0, CPR, chunk_body, (z, z))
            out_v[i, pl.ds(0, 16)] = a0 * inv_l
            out_v[i, pl.ds(16, 16)] = a1 * inv_l
            return carry

        lax.fori_loop(0, BPW, row_body, 0)
        pltpu.sync_copy(out_v, out_hbm.at[pl.ds(wid * BPW, BPW)])

    return pool


_pool = _make_pool()

BN = 2048
GRID_N = (VOCAB + BN - 1) // BN


def _matmul_body(w_ref, b_ref, x_ref, o_ref):
    lhs = jnp.concatenate([w_ref[...], b_ref[...]], axis=0)
    rhs = jnp.concatenate(
        [x_ref[...], jnp.ones((B, 1), jnp.float32)], axis=1
    )
    o_ref[...] = lax.dot_general(
        lhs, rhs, (((0,), (1,)), ((), ())),
        preferred_element_type=jnp.float32,
    )


def _matmul_t(w, b2, pooled):
    return pl.pallas_call(
        _matmul_body,
        grid=(GRID_N,),
        in_specs=[
            pl.BlockSpec((EMBED, BN), lambda n: (0, n)),
            pl.BlockSpec((1, BN), lambda n: (0, n)),
            pl.BlockSpec((B, EMBED), lambda n: (0, 0)),
        ],
        out_specs=pl.BlockSpec((BN, B), lambda n: (n, 0)),
        out_shape=jax.ShapeDtypeStruct((VOCAB, B), jnp.float32),
    )(w, b2, pooled)


def kernel(input_ids, embed_table, W, b):
    ids3 = input_ids.reshape(NW, NCHUNK, CHUNK)
    pooled = _pool(ids3, embed_table)
    logits_t = _matmul_t(W, b.reshape(1, VOCAB), pooled)
    return jnp.transpose(logits_t)[:, None, :]


# speedup vs baseline: 2.2903x; 1.0007x over previous
import functools

import jax
import jax.numpy as jnp
from jax import lax
from jax.experimental import pallas as pl
from jax.experimental.pallas import tpu as pltpu
from jax.experimental.pallas import tpu_sc as plsc

VOCAB = 100000
EMBED = 32
B = 1024
L = 200

NC = 2
NS = 16
NW = NC * NS
BPW = B // NW
CHUNK = 100
CPR = L // CHUNK
NCHUNK = BPW * CPR


def _make_pool():
    mesh = plsc.VectorSubcoreMesh(core_axis_name="c", subcore_axis_name="s")

    @functools.partial(
        pl.kernel,
        mesh=mesh,
        compiler_params=pltpu.CompilerParams(use_tc_tiling_on_sc=False),
        out_type=jax.ShapeDtypeStruct((B, EMBED), jnp.float32),
        scratch_types=[
            pltpu.VMEM((NCHUNK, CHUNK), jnp.int32),
            pltpu.VMEM((CHUNK, EMBED), jnp.float32),
            pltpu.VMEM((BPW, EMBED), jnp.float32),
            pltpu.SemaphoreType.DMA,
        ],
    )
    def pool(ids_hbm, table_hbm, out_hbm, idx_v, rows_v, out_v, sem):
        wid = lax.axis_index("s") * NC + lax.axis_index("c")
        pltpu.sync_copy(ids_hbm.at[wid], idx_v)
        inv_l = jnp.float32(1.0 / L)

        def row_body(i, carry):
            def chunk_body(k, accs):
                a0, a1 = accs
                pltpu.async_copy(
                    table_hbm.at[idx_v.at[i * CPR + k]], rows_v, sem
                ).wait()

                def tok_body(t, accs2):
                    b0, b1 = accs2
                    return (b0 + rows_v[t, pl.ds(0, 16)],
                            b1 + rows_v[t, pl.ds(16, 16)])

                return lax.fori_loop(0, CHUNK, tok_body, (a0, a1), unroll=10)

            z = jnp.zeros((16,), jnp.float32)
            a0, a1 = lax.fori_loop(0, CPR, chunk_body, (z, z))
            out_v[i, pl.ds(0, 16)] = a0 * inv_l
            out_v[i, pl.ds(16, 16)] = a1 * inv_l
            return carry

        lax.fori_loop(0, BPW, row_body, 0)
        pltpu.sync_copy(out_v, out_hbm.at[pl.ds(wid * BPW, BPW)])

    return pool


_pool = _make_pool()

BN = 4096
GRID_N = (VOCAB + BN - 1) // BN


def _matmul_body(w_ref, b_ref, x_ref, o_ref):
    lhs = jnp.concatenate([w_ref[...], b_ref[...]], axis=0)
    rhs = jnp.concatenate(
        [x_ref[...], jnp.ones((B, 1), jnp.float32)], axis=1
    )
    o_ref[...] = lax.dot_general(
        lhs, rhs, (((0,), (1,)), ((), ())),
        preferred_element_type=jnp.float32,
    )


def _matmul_t(w, b2, pooled):
    return pl.pallas_call(
        _matmul_body,
        grid=(GRID_N,),
        in_specs=[
            pl.BlockSpec((EMBED, BN), lambda n: (0, n)),
            pl.BlockSpec((1, BN), lambda n: (0, n)),
            pl.BlockSpec((B, EMBED), lambda n: (0, 0)),
        ],
        out_specs=pl.BlockSpec((BN, B), lambda n: (n, 0)),
        out_shape=jax.ShapeDtypeStruct((VOCAB, B), jnp.float32),
    )(w, b2, pooled)


def kernel(input_ids, embed_table, W, b):
    ids3 = input_ids.reshape(NW, NCHUNK, CHUNK)
    pooled = _pool(ids3, embed_table)
    logits_t = _matmul_t(W, b.reshape(1, VOCAB), pooled)
    return jnp.transpose(logits_t)[:, None, :]


# speedup vs baseline: 2.5529x; 1.1147x over previous
import functools

import jax
import jax.numpy as jnp
from jax import lax
from jax.experimental import pallas as pl
from jax.experimental.pallas import tpu as pltpu
from jax.experimental.pallas import tpu_sc as plsc

VOCAB = 100000
EMBED = 32
B = 1024
L = 200

NC = 2
NS = 16
NW = NC * NS
BPW = B // NW
CHUNK = 100
CPR = L // CHUNK
NCHUNK = BPW * CPR


def _make_pool():
    mesh = plsc.VectorSubcoreMesh(core_axis_name="c", subcore_axis_name="s")

    @functools.partial(
        pl.kernel,
        mesh=mesh,
        compiler_params=pltpu.CompilerParams(use_tc_tiling_on_sc=False),
        out_type=jax.ShapeDtypeStruct((B, EMBED), jnp.float32),
        scratch_types=[
            pltpu.VMEM((NCHUNK, CHUNK), jnp.int32),
            pltpu.VMEM((CHUNK, EMBED), jnp.float32),
            pltpu.VMEM((CHUNK, EMBED), jnp.float32),
            pltpu.VMEM((BPW, EMBED), jnp.float32),
            pltpu.SemaphoreType.DMA,
            pltpu.SemaphoreType.DMA,
        ],
    )
    def pool(ids_hbm, table_hbm, out_hbm, idx_v, rows_a, rows_b, out_v,
             sem_a, sem_b):
        wid = lax.axis_index("s") * NC + lax.axis_index("c")
        pltpu.sync_copy(ids_hbm.at[wid], idx_v)
        inv_l = jnp.float32(1.0 / L)

        def acc_chunk(rows_v, accs):
            def tok_body(t, accs2):
                b0, b1 = accs2
                return (b0 + rows_v[t, pl.ds(0, 16)],
                        b1 + rows_v[t, pl.ds(16, 16)])

            return lax.fori_loop(0, CHUNK, tok_body, accs, unroll=10)

        pltpu.async_copy(table_hbm.at[idx_v.at[0]], rows_a, sem_a)

        def row_body(i, carry):
            pltpu.async_copy(table_hbm.at[idx_v.at[2 * i + 1]], rows_b, sem_b)
            pltpu.make_async_copy(table_hbm.at[idx_v.at[0]], rows_a,
                                  sem_a).wait()
            z = jnp.zeros((16,), jnp.float32)
            a0, a1 = acc_chunk(rows_a, (z, z))

            @pl.when(i < BPW - 1)
            def _():
                pltpu.async_copy(table_hbm.at[idx_v.at[2 * i + 2]], rows_a,
                                 sem_a)

            pltpu.make_async_copy(table_hbm.at[idx_v.at[0]], rows_b,
                                  sem_b).wait()
            a0, a1 = acc_chunk(rows_b, (a0, a1))
            out_v[i, pl.ds(0, 16)] = a0 * inv_l
            out_v[i, pl.ds(16, 16)] = a1 * inv_l
            return carry

        lax.fori_loop(0, BPW, row_body, 0)
        pltpu.sync_copy(out_v, out_hbm.at[pl.ds(wid * BPW, BPW)])

    return pool


_pool = _make_pool()

BN = 4096
GRID_N = (VOCAB + BN - 1) // BN


def _matmul_body(w_ref, b_ref, x_ref, o_ref):
    lhs = jnp.concatenate([w_ref[...], b_ref[...]], axis=0)
    rhs = jnp.concatenate(
        [x_ref[...], jnp.ones((B, 1), jnp.float32)], axis=1
    )
    o_ref[...] = lax.dot_general(
        lhs, rhs, (((0,), (1,)), ((), ())),
        preferred_element_type=jnp.float32,
    )


def _matmul_t(w, b2, pooled):
    return pl.pallas_call(
        _matmul_body,
        grid=(GRID_N,),
        in_specs=[
            pl.BlockSpec((EMBED, BN), lambda n: (0, n)),
            pl.BlockSpec((1, BN), lambda n: (0, n)),
            pl.BlockSpec((B, EMBED), lambda n: (0, 0)),
        ],
        out_specs=pl.BlockSpec((BN, B), lambda n: (n, 0)),
        out_shape=jax.ShapeDtypeStruct((VOCAB, B), jnp.float32),
    )(w, b2, pooled)


def kernel(input_ids, embed_table, W, b):
    ids3 = input_ids.reshape(NW, NCHUNK, CHUNK)
    pooled = _pool(ids3, embed_table)
    logits_t = _matmul_t(W, b.reshape(1, VOCAB), pooled)
    return jnp.transpose(logits_t)[:, None, :]
